# baseline (speedup 1.0000x reference)
import jax
import jax.numpy as jnp
from jax.experimental import pallas as pl
from jax.experimental.pallas import tpu as pltpu

_Q = 64
_D = 768
_K = 100000
_TOPK = 5
_KB = 5000
_NB = _K // _KB


def _rsqrt_refined(x):
    r = jax.lax.rsqrt(x)
    return r * (1.5 - 0.5 * x * r * r)


def _fused(q_ref, k_ref, vals_ref, idx_ref):
    j = pl.program_id(0)

    @pl.when(j == 0)
    def _init():
        vals_ref[...] = jnp.full((_Q, _TOPK), -jnp.inf, jnp.float32)
        idx_ref[...] = jnp.zeros((_Q, _TOPK), jnp.int32)

    q = q_ref[...]
    kb = k_ref[...]
    rqn = _rsqrt_refined(jnp.sum(q * q, axis=1, keepdims=True))
    s = jax.lax.dot_general(q, kb, (((1,), (1,)), ((), ())),
                            preferred_element_type=jnp.float32)
    kb2 = kb * kb
    ones_r = jnp.ones((8, _D), jnp.float32)
    kn2 = jax.lax.dot_general(ones_r, kb2, (((1,), (1,)), ((), ())),
                              precision=jax.lax.Precision.HIGHEST,
                              preferred_element_type=jnp.float32)
    rkn = _rsqrt_refined(kn2[0:1, :])
    lane = jax.lax.broadcasted_iota(jnp.int32, (_Q, _KB), 1)
    s = jnp.where(lane < _KB, s * rqn * rkn, -jnp.inf)
    bvals, bidx = [], []
    for _ in range(_TOPK):
        m = jnp.max(s, axis=1, keepdims=True)
        d = jnp.where(s == m, lane, _KB)
        pos = jnp.min(d, axis=1, keepdims=True)
        bvals.append(m)
        bidx.append(j * _KB + pos)
        s = jnp.where(d < _KB, -jnp.inf, s)

    cand_v = jnp.concatenate([vals_ref[...]] + bvals, axis=1)
    cand_i = jnp.concatenate([idx_ref[...]] + bidx, axis=1)
    lane2 = jax.lax.broadcasted_iota(jnp.int32, (_Q, 2 * _TOPK), 1)
    cand_v = jnp.where(lane2 < 2 * _TOPK, cand_v, -jnp.inf)
    out_v, out_i = [], []
    for _ in range(_TOPK):
        m = jnp.max(cand_v, axis=1, keepdims=True)
        pos = jnp.min(jnp.where(cand_v == m, lane2, 2 * _TOPK), axis=1,
                      keepdims=True)
        out_v.append(m)
        out_i.append(jnp.sum(jnp.where(lane2 == pos, cand_i, 0), axis=1,
                             keepdims=True))
        cand_v = jnp.where(lane2 == pos, -jnp.inf, cand_v)
    vals_ref[...] = jnp.concatenate(out_v, axis=1)
    idx_ref[...] = jnp.concatenate(out_i, axis=1)


def kernel(queries, keys, k):
    vals, idx = pl.pallas_call(
        _fused,
        grid=(_NB,),
        in_specs=[
            pl.BlockSpec((_Q, _D), lambda j: (0, 0)),
            pl.BlockSpec((_KB, _D), lambda j: (j, 0)),
        ],
        out_specs=[
            pl.BlockSpec((_Q, _TOPK), lambda j: (0, 0)),
            pl.BlockSpec((_Q, _TOPK), lambda j: (0, 0)),
        ],
        out_shape=[
            jax.ShapeDtypeStruct((_Q, _TOPK), jnp.float32),
            jax.ShapeDtypeStruct((_Q, _TOPK), jnp.int32),
        ],
        compiler_params=pltpu.CompilerParams(
            dimension_semantics=("arbitrary",),
        ),
    )(queries, keys)
    return vals, idx


# speedup vs baseline: 3.0204x; 3.0204x over previous
import jax
import jax.numpy as jnp
from jax.experimental import pallas as pl
from jax.experimental.pallas import tpu as pltpu

_Q = 64
_D = 768
_K = 100000
_TOPK = 5
_KB = 5000
_NB = _K // _KB


def _rsqrt_refined(x):
    r = jax.lax.rsqrt(x)
    return r * (1.5 - 0.5 * x * r * r)


def _fused(q_ref, k_ref, vals_ref, idx_ref):
    j = pl.program_id(0)

    @pl.when(j == 0)
    def _init():
        vals_ref[...] = jnp.full((_Q, _TOPK), -jnp.inf, jnp.float32)
        idx_ref[...] = jnp.zeros((_Q, _TOPK), jnp.int32)

    q = q_ref[...]
    kb = k_ref[...]
    rqn = _rsqrt_refined(jnp.sum(q * q, axis=1, keepdims=True))
    s = jax.lax.dot_general(q, kb, (((1,), (1,)), ((), ())),
                            preferred_element_type=jnp.float32)
    kn2 = jnp.sum(kb * kb, axis=1)[None, :]
    rkn = _rsqrt_refined(kn2)
    lane = jax.lax.broadcasted_iota(jnp.int32, (_Q, _KB), 1)
    s = jnp.where(lane < _KB, s * rqn * rkn, -jnp.inf)
    bvals, bidx = [], []
    for _ in range(_TOPK):
        m = jnp.max(s, axis=1, keepdims=True)
        d = jnp.where(s == m, lane, _KB)
        pos = jnp.min(d, axis=1, keepdims=True)
        bvals.append(m)
        bidx.append(j * _KB + pos)
        s = jnp.where(d < _KB, -jnp.inf, s)

    cand_v = jnp.concatenate([vals_ref[...]] + bvals, axis=1)
    cand_i = jnp.concatenate([idx_ref[...]] + bidx, axis=1)
    lane2 = jax.lax.broadcasted_iota(jnp.int32, (_Q, 2 * _TOPK), 1)
    cand_v = jnp.where(lane2 < 2 * _TOPK, cand_v, -jnp.inf)
    out_v, out_i = [], []
    for _ in range(_TOPK):
        m = jnp.max(cand_v, axis=1, keepdims=True)
        pos = jnp.min(jnp.where(cand_v == m, lane2, 2 * _TOPK), axis=1,
                      keepdims=True)
        out_v.append(m)
        out_i.append(jnp.sum(jnp.where(lane2 == pos, cand_i, 0), axis=1,
                             keepdims=True))
        cand_v = jnp.where(lane2 == pos, -jnp.inf, cand_v)
    vals_ref[...] = jnp.concatenate(out_v, axis=1)
    idx_ref[...] = jnp.concatenate(out_i, axis=1)


def kernel(queries, keys, k):
    vals, idx = pl.pallas_call(
        _fused,
        grid=(_NB,),
        in_specs=[
            pl.BlockSpec((_Q, _D), lambda j: (0, 0)),
            pl.BlockSpec((_KB, _D), lambda j: (j, 0)),
        ],
        out_specs=[
            pl.BlockSpec((_Q, _TOPK), lambda j: (0, 0)),
            pl.BlockSpec((_Q, _TOPK), lambda j: (0, 0)),
        ],
        out_shape=[
            jax.ShapeDtypeStruct((_Q, _TOPK), jnp.float32),
            jax.ShapeDtypeStruct((_Q, _TOPK), jnp.int32),
        ],
        compiler_params=pltpu.CompilerParams(
            dimension_semantics=("arbitrary",),
        ),
    )(queries, keys)
    return vals, idx
